# baseline (speedup 1.0000x reference)
import functools

import jax
import jax.numpy as jnp
from jax import lax
from jax.experimental import pallas as pl
from jax.experimental.pallas import tpu as pltpu
from jax.experimental.pallas import tpu_sc as plsc

B = 16384
D = 16
L = 16
NC = 2
NS = 16
NW = NC * NS
BPW = B // NW
CHUNK = 128
NCH = BPW // CHUNK
GPC = CHUNK // L
XROWS = BPW * D // 128


def _sc_partials(x2, xn, yn, ht2, vt2):
    mesh = plsc.VectorSubcoreMesh(core_axis_name="c", subcore_axis_name="s")

    @functools.partial(
        pl.kernel,
        mesh=mesh,
        compiler_params=pltpu.CompilerParams(
            needs_layout_passes=False, use_tc_tiling_on_sc=True),
        out_type=jax.ShapeDtypeStruct((NW, D), jnp.float32),
        scratch_types=[
            pltpu.VMEM((BPW,), jnp.int32),
            pltpu.VMEM((BPW,), jnp.int32),
            pltpu.VMEM((BPW,), jnp.int32),
            pltpu.VMEM((BPW,), jnp.int32),
            pltpu.VMEM((CHUNK, 128), jnp.float32),
            pltpu.VMEM((CHUNK, 128), jnp.float32),
            pltpu.VMEM((CHUNK, 128), jnp.float32),
            pltpu.VMEM((CHUNK, 128), jnp.float32),
            pltpu.VMEM((XROWS, 128), jnp.float32),
            pltpu.VMEM((D,), jnp.float32),
            pltpu.SemaphoreType.DMA,
            pltpu.SemaphoreType.DMA,
        ],
    )
    def k(x_hbm, xn_hbm, yn_hbm, ht_hbm, vt_hbm, out_hbm,
          idx_u, idx_v, blk_u, blk_v, bu0, bu1, bv0, bv1, x_v, acc_v,
          sem_u, sem_v):
        wid = lax.axis_index("s") * NC + lax.axis_index("c")
        base = wid * BPW
        iota = lax.iota(jnp.int32, L)
        bufs_u = (bu0, bu1)
        bufs_v = (bv0, bv1)

        pltpu.sync_copy(xn_hbm.at[pl.ds(base, BPW)], idx_u)
        pltpu.sync_copy(yn_hbm.at[pl.ds(base, BPW)], idx_v)

        def prep(g, carry):
            off = g * L
            idx_u[pl.ds(off, L)]
            blk_u[pl.ds(off, L)] = lax.shift_right_logical(
                idx_u[pl.ds(off, L)], jnp.int32(3))
            blk_v[pl.ds(off, L)] = lax.shift_right_logical(
                idx_v[pl.ds(off, L)], jnp.int32(3))
            return carry
        lax.fori_loop(jnp.int32(0), jnp.int32(BPW // L), prep, 0)

        def fire(c):
            cb = c % 2
            pltpu.async_copy(ht_hbm.at[blk_u.at[pl.ds(c * CHUNK, CHUNK)]],
                             bufs_u[cb], sem_u)
            pltpu.async_copy(vt_hbm.at[blk_v.at[pl.ds(c * CHUNK, CHUNK)]],
                             bufs_v[cb], sem_v)

        def wait(c):
            cb = c % 2
            pltpu.make_async_copy(ht_hbm.at[blk_u.at[pl.ds(c * CHUNK, CHUNK)]],
                                  bufs_u[cb], sem_u).wait()
            pltpu.make_async_copy(vt_hbm.at[blk_v.at[pl.ds(c * CHUNK, CHUNK)]],
                                  bufs_v[cb], sem_v).wait()

        fire(0)
        pltpu.sync_copy(x_hbm.at[pl.ds(wid * XROWS, XROWS)], x_v)

        zero = jnp.zeros((L,), jnp.float32)
        ts = (zero,) * D
        for c in range(NCH):
            if c + 1 < NCH:
                fire(c + 1)
            wait(c)
            bu = bufs_u[c % 2]
            bv = bufs_v[c % 2]

            def group(g, ts, c=c, bu=bu, bv=bv):
                aoff = c * CHUNK + g * L
                rloc = g * L + iota
                rabs = aoff + iota
                iu = idx_u[pl.ds(aoff, L)]
                iv = idx_v[pl.ds(aoff, L)]
                ju = (iu & 7) * L
                jv = (iv & 7) * L
                xd0 = lax.shift_right_logical(rabs, jnp.int32(3))
                xd1 = (rabs & 7) * L
                s = zero
                for kk in range(D):
                    uc = plsc.load_gather(bu, [rloc, ju + kk])
                    xc = plsc.load_gather(x_v, [xd0, xd1 + kk])
                    s = s + uc * xc
                new_ts = []
                for kk in range(D):
                    vc = plsc.load_gather(bv, [rloc, jv + kk])
                    new_ts.append(ts[kk] + s * vc)
                return tuple(new_ts)

            ts = lax.fori_loop(jnp.int32(0), jnp.int32(GPC), group, ts)

        acc = jnp.zeros((L,), jnp.float32)
        for kk in range(D):
            onehot = (iota == kk).astype(jnp.float32)
            acc = acc + jnp.sum(ts[kk]) * onehot
        acc_v[...] = acc
        pltpu.sync_copy(acc_v, out_hbm.at[wid])

    return k(x2, xn, yn, ht2, vt2)


def _tc_finish_body(y2_ref, p_ref, o_ref):
    t = jnp.sum(p_ref[...], axis=0)
    j = lax.broadcasted_iota(jnp.int32, (128, 8), 0)
    jj = lax.broadcasted_iota(jnp.int32, (128, 8), 1)
    tt = jnp.tile(t, 8)
    sel = jnp.where((j >> 4) == jj, tt[:, None], 0.0)
    o_ref[...] = jnp.dot(y2_ref[...], sel,
                         preferred_element_type=jnp.float32)


def _tc_finish(y2, partials):
    return pl.pallas_call(
        _tc_finish_body,
        out_shape=jax.ShapeDtypeStruct((B // 8, 8), jnp.float32),
    )(y2, partials)


def kernel(x, y, x_nodes, y_nodes, human_table, virus_table):
    xn = x_nodes.astype(jnp.int32)
    yn = y_nodes.astype(jnp.int32)
    ht2 = human_table.reshape(-1, 128)
    vt2 = virus_table.reshape(-1, 128)
    x2 = x.reshape(-1, 128)
    y2 = y.reshape(-1, 128)
    partials = _sc_partials(x2, xn, yn, ht2, vt2)
    out = _tc_finish(y2, partials).reshape(B)
    return out.astype(jnp.float64)
